# baseline (speedup 1.0000x reference)
import jax
import jax.numpy as jnp
from jax.experimental import pallas as pl
from jax.experimental.pallas import tpu as pltpu

_NFEAT, _NHID, _NMETA, _DIM_MP, _NCLASS = 128, 64, 3, 32, 8
_ALPHA = 0.2
_N = 4096
_BM = 1024
_NB = _N // _BM


def _layer1_body(x_ref, adj_ref, w1_ref, b1_ref, h1_ref, q_ref, y1_s):
    m = pl.program_id(0)
    i = pl.program_id(1)
    adj = adj_ref[0]

    @pl.when(i == 0)
    def _compute_y1():
        y1_s[...] = jnp.dot(x_ref[...], w1_ref[m],
                            preferred_element_type=jnp.float32
                            ).astype(jnp.bfloat16)

    h = jnp.dot(adj.astype(jnp.bfloat16), y1_s[...],
                preferred_element_type=jnp.float32)
    h1_ref[0] = jnp.maximum(h + b1_ref[m], 0.0).astype(jnp.bfloat16)
    q_ref[0] = jnp.round((adj - 0.5) * 254.0).astype(jnp.int8)


def _layer2_body(q_ref, h1_ref, w2_ref, b2_ref, a_ref, wlin_ref, blin_ref,
                 out_ref, qhi_s, qlo_s, shi_s, slo_s, cs_s, e0_s, e1_s):
    m = pl.program_id(0)
    i = pl.program_id(1)
    row0 = i * _BM
    q = q_ref[0]

    @pl.when(i == 0)
    def _compute_y2():
        y2 = jnp.dot(h1_ref[0], w2_ref[m],
                     preferred_element_type=jnp.float32)
        s_hi = jnp.maximum(jnp.max(jnp.abs(y2), axis=0, keepdims=True),
                           1e-30) / 127.0
        r_hi = jnp.round(y2 / s_hi)
        qhi_s[...] = r_hi.astype(jnp.int8)
        resid = y2 - s_hi * r_hi
        s_lo = jnp.maximum(jnp.max(jnp.abs(resid), axis=0, keepdims=True),
                           1e-30) / 127.0
        qlo_s[...] = jnp.round(resid / s_lo).astype(jnp.int8)
        shi_s[...] = s_hi / 254.0
        slo_s[...] = s_lo / 254.0
        cs_s[...] = 0.5 * jnp.sum(y2, axis=0, keepdims=True)

    acc_hi = jnp.dot(q, qhi_s[...], preferred_element_type=jnp.int32)
    acc_lo = jnp.dot(q, qlo_s[...], preferred_element_type=jnp.int32)
    e = (acc_hi.astype(jnp.float32) * shi_s[...]
         + acc_lo.astype(jnp.float32) * slo_s[...]
         + cs_s[...] + b2_ref[m])
    e = jnp.maximum(e, 0.0)

    @pl.when(m == 0)
    def _store0():
        e0_s[pl.ds(row0, _BM), :] = e

    @pl.when(m == 1)
    def _store1():
        e1_s[pl.ds(row0, _BM), :] = e

    @pl.when(m == _NMETA - 1)
    def _attend():
        e0 = e0_s[pl.ds(row0, _BM), :]
        e1 = e1_s[pl.ds(row0, _BM), :]
        a_v = a_ref[...]
        s0 = jnp.dot(e0, a_v, preferred_element_type=jnp.float32)
        s1 = jnp.dot(e1, a_v, preferred_element_type=jnp.float32)
        s2 = jnp.dot(e, a_v, preferred_element_type=jnp.float32)
        s0 = jnp.where(s0 > 0, s0, _ALPHA * s0)
        s1 = jnp.where(s1 > 0, s1, _ALPHA * s1)
        s2 = jnp.where(s2 > 0, s2, _ALPHA * s2)
        mx = jnp.maximum(s0, jnp.maximum(s1, s2))
        x0 = jnp.exp(s0 - mx)
        x1 = jnp.exp(s1 - mx)
        x2 = jnp.exp(s2 - mx)
        comb = (x0 * e0 + x1 * e1 + x2 * e) / (x0 + x1 + x2)
        logits = jnp.dot(comb, wlin_ref[...],
                         preferred_element_type=jnp.float32)
        logits = jnp.maximum(logits + blin_ref[...], 0.0)
        zmax = jnp.max(logits, axis=1, keepdims=True)
        z = logits - zmax
        out_ref[...] = z - jnp.log(jnp.sum(jnp.exp(z), axis=1,
                                           keepdims=True))


def kernel(input, adjs, W1, b1, W2, b2, a, Wlin, blin):
    b1r = b1.reshape(_NMETA, 1, _NHID)
    b2r = b2.reshape(_NMETA, 1, _DIM_MP)
    a_r = a.reshape(_DIM_MP, 1)
    blin_r = blin.reshape(1, _NCLASS)
    w2_bf = W2.astype(jnp.bfloat16)

    h1, q = pl.pallas_call(
        _layer1_body,
        grid=(_NMETA, _NB),
        in_specs=[
            pl.BlockSpec((_N, _NFEAT), lambda m, i: (0, 0)),
            pl.BlockSpec((1, _BM, _N), lambda m, i: (m, i, 0)),
            pl.BlockSpec((_NMETA, _NFEAT, _NHID), lambda m, i: (0, 0, 0)),
            pl.BlockSpec((_NMETA, 1, _NHID), lambda m, i: (0, 0, 0)),
        ],
        out_specs=[
            pl.BlockSpec((1, _BM, _NHID), lambda m, i: (m, i, 0)),
            pl.BlockSpec((1, _BM, _N), lambda m, i: (m, i, 0)),
        ],
        out_shape=[
            jax.ShapeDtypeStruct((_NMETA, _N, _NHID), jnp.bfloat16),
            jax.ShapeDtypeStruct((_NMETA, _N, _N), jnp.int8),
        ],
        compiler_params=pltpu.CompilerParams(
            vmem_limit_bytes=62 * 1024 * 1024,
        ),
        scratch_shapes=[
            pltpu.VMEM((_N, _NHID), jnp.bfloat16),
        ],
    )(input, adjs, W1, b1r)

    return pl.pallas_call(
        _layer2_body,
        grid=(_NMETA, _NB),
        in_specs=[
            pl.BlockSpec((1, _BM, _N), lambda m, i: (m, i, 0)),
            pl.BlockSpec((1, _N, _NHID), lambda m, i: (m, 0, 0)),
            pl.BlockSpec((_NMETA, _NHID, _DIM_MP), lambda m, i: (0, 0, 0)),
            pl.BlockSpec((_NMETA, 1, _DIM_MP), lambda m, i: (0, 0, 0)),
            pl.BlockSpec((_DIM_MP, 1), lambda m, i: (0, 0)),
            pl.BlockSpec((_DIM_MP, _NCLASS), lambda m, i: (0, 0)),
            pl.BlockSpec((1, _NCLASS), lambda m, i: (0, 0)),
        ],
        out_specs=pl.BlockSpec((_BM, _NCLASS), lambda m, i: (i, 0)),
        out_shape=jax.ShapeDtypeStruct((_N, _NCLASS), jnp.float32),
        compiler_params=pltpu.CompilerParams(
            vmem_limit_bytes=62 * 1024 * 1024,
        ),
        scratch_shapes=[
            pltpu.VMEM((_N, _DIM_MP), jnp.int8),
            pltpu.VMEM((_N, _DIM_MP), jnp.int8),
            pltpu.VMEM((1, _DIM_MP), jnp.float32),
            pltpu.VMEM((1, _DIM_MP), jnp.float32),
            pltpu.VMEM((1, _DIM_MP), jnp.float32),
            pltpu.VMEM((_N, _DIM_MP), jnp.float32),
            pltpu.VMEM((_N, _DIM_MP), jnp.float32),
        ],
    )(q, h1, w2_bf, b2r, a_r, Wlin, blin_r)


# speedup vs baseline: 1.2229x; 1.2229x over previous
import jax
import jax.numpy as jnp
from jax.experimental import pallas as pl
from jax.experimental.pallas import tpu as pltpu

_NFEAT, _NHID, _NMETA, _DIM_MP, _NCLASS = 128, 64, 3, 32, 8
_ALPHA = 0.2
_N = 4096
_BM = 1024
_NB = _N // _BM
_BM2 = 2048
_NB2 = _N // _BM2


def _layer1_body(x_ref, adj_ref, w1_ref, b1_ref, h1_ref, q_ref, y1_s):
    m = pl.program_id(0)
    i = pl.program_id(1)
    adj = adj_ref[0]

    @pl.when(i == 0)
    def _compute_y1():
        y1_s[...] = jnp.dot(x_ref[...], w1_ref[m],
                            preferred_element_type=jnp.float32
                            ).astype(jnp.bfloat16)

    h = jnp.dot(adj.astype(jnp.bfloat16), y1_s[...],
                preferred_element_type=jnp.float32)
    h1_ref[0] = jnp.maximum(h + b1_ref[m], 0.0).astype(jnp.bfloat16)
    q_ref[0] = jnp.round(adj * 254.0 - 127.0).astype(jnp.int8)


def _layer2_body(q_ref, h1_ref, w2_ref, b2_ref, a_ref, wlin_ref, blin_ref,
                 out_ref, y2b_s, cs_s, e0_s, e1_s):
    m = pl.program_id(0)
    i = pl.program_id(1)
    row0 = i * _BM2

    @pl.when(i == 0)
    def _compute_y2():
        y2 = jnp.dot(h1_ref[0], w2_ref[m],
                     preferred_element_type=jnp.float32)
        y2b_s[...] = (y2 * (1.0 / 254.0)).astype(jnp.bfloat16)
        cs_s[...] = 0.5 * jnp.sum(y2, axis=0, keepdims=True) + b2_ref[m]

    e = jnp.dot(q_ref[0].astype(jnp.bfloat16), y2b_s[...],
                preferred_element_type=jnp.float32)
    e = jnp.maximum(e + cs_s[...], 0.0)

    @pl.when(m == 0)
    def _store0():
        e0_s[pl.ds(row0, _BM2), :] = e

    @pl.when(m == 1)
    def _store1():
        e1_s[pl.ds(row0, _BM2), :] = e

    @pl.when(m == _NMETA - 1)
    def _attend():
        e0 = e0_s[pl.ds(row0, _BM2), :]
        e1 = e1_s[pl.ds(row0, _BM2), :]
        a_v = a_ref[...]
        s0 = jnp.dot(e0, a_v, preferred_element_type=jnp.float32)
        s1 = jnp.dot(e1, a_v, preferred_element_type=jnp.float32)
        s2 = jnp.dot(e, a_v, preferred_element_type=jnp.float32)
        s0 = jnp.where(s0 > 0, s0, _ALPHA * s0)
        s1 = jnp.where(s1 > 0, s1, _ALPHA * s1)
        s2 = jnp.where(s2 > 0, s2, _ALPHA * s2)
        mx = jnp.maximum(s0, jnp.maximum(s1, s2))
        x0 = jnp.exp(s0 - mx)
        x1 = jnp.exp(s1 - mx)
        x2 = jnp.exp(s2 - mx)
        comb = (x0 * e0 + x1 * e1 + x2 * e) / (x0 + x1 + x2)
        logits = jnp.dot(comb, wlin_ref[...],
                         preferred_element_type=jnp.float32)
        logits = jnp.maximum(logits + blin_ref[...], 0.0)
        zmax = jnp.max(logits, axis=1, keepdims=True)
        z = logits - zmax
        out_ref[...] = z - jnp.log(jnp.sum(jnp.exp(z), axis=1,
                                           keepdims=True))


def kernel(input, adjs, W1, b1, W2, b2, a, Wlin, blin):
    b1r = b1.reshape(_NMETA, 1, _NHID)
    b2r = b2.reshape(_NMETA, 1, _DIM_MP)
    a_r = a.reshape(_DIM_MP, 1)
    blin_r = blin.reshape(1, _NCLASS)
    w2_bf = W2.astype(jnp.bfloat16)

    h1, q = pl.pallas_call(
        _layer1_body,
        grid=(_NMETA, _NB),
        in_specs=[
            pl.BlockSpec((_N, _NFEAT), lambda m, i: (0, 0)),
            pl.BlockSpec((1, _BM, _N), lambda m, i: (m, i, 0)),
            pl.BlockSpec((_NMETA, _NFEAT, _NHID), lambda m, i: (0, 0, 0)),
            pl.BlockSpec((_NMETA, 1, _NHID), lambda m, i: (0, 0, 0)),
        ],
        out_specs=[
            pl.BlockSpec((1, _BM, _NHID), lambda m, i: (m, i, 0)),
            pl.BlockSpec((1, _BM, _N), lambda m, i: (m, i, 0)),
        ],
        out_shape=[
            jax.ShapeDtypeStruct((_NMETA, _N, _NHID), jnp.bfloat16),
            jax.ShapeDtypeStruct((_NMETA, _N, _N), jnp.int8),
        ],
        compiler_params=pltpu.CompilerParams(
            vmem_limit_bytes=62 * 1024 * 1024,
        ),
        scratch_shapes=[
            pltpu.VMEM((_N, _NHID), jnp.bfloat16),
        ],
    )(input, adjs, W1, b1r)

    return pl.pallas_call(
        _layer2_body,
        grid=(_NMETA, _NB2),
        in_specs=[
            pl.BlockSpec((1, _BM2, _N), lambda m, i: (m, i, 0)),
            pl.BlockSpec((1, _N, _NHID), lambda m, i: (m, 0, 0)),
            pl.BlockSpec((_NMETA, _NHID, _DIM_MP), lambda m, i: (0, 0, 0)),
            pl.BlockSpec((_NMETA, 1, _DIM_MP), lambda m, i: (0, 0, 0)),
            pl.BlockSpec((_DIM_MP, 1), lambda m, i: (0, 0)),
            pl.BlockSpec((_DIM_MP, _NCLASS), lambda m, i: (0, 0)),
            pl.BlockSpec((1, _NCLASS), lambda m, i: (0, 0)),
        ],
        out_specs=pl.BlockSpec((_BM2, _NCLASS), lambda m, i: (i, 0)),
        out_shape=jax.ShapeDtypeStruct((_N, _NCLASS), jnp.float32),
        compiler_params=pltpu.CompilerParams(
            vmem_limit_bytes=62 * 1024 * 1024,
        ),
        scratch_shapes=[
            pltpu.VMEM((_N, _DIM_MP), jnp.bfloat16),
            pltpu.VMEM((1, _DIM_MP), jnp.float32),
            pltpu.VMEM((_N, _DIM_MP), jnp.float32),
            pltpu.VMEM((_N, _DIM_MP), jnp.float32),
        ],
    )(q, h1, w2_bf, b2r, a_r, Wlin, blin_r)


# speedup vs baseline: 1.4388x; 1.1765x over previous
import jax
import jax.numpy as jnp
from jax.experimental import pallas as pl
from jax.experimental.pallas import tpu as pltpu

_NFEAT, _NHID, _NMETA, _DIM_MP, _NCLASS = 128, 64, 3, 32, 8
_ALPHA = 0.2
_N = 4096
_BM = 512
_NB = _N // _BM


def _hingcn_body(x_ref, adj_ref, w1_ref, b1_ref, w2_ref, b2_ref, a_ref,
                 wlin_ref, blin_ref, out_ref,
                 abf_s, y1_s, h1_s, y2_s, e0_s, e1_s):
    m = pl.program_id(0)
    layer = pl.program_id(1)
    i = pl.program_id(2)
    row0 = i * _BM

    @pl.when(layer == 0)
    def _layer1():
        @pl.when(i == 0)
        def _compute_y1():
            y1_s[...] = jnp.dot(x_ref[...], w1_ref[m],
                                preferred_element_type=jnp.float32
                                ).astype(jnp.bfloat16)

        abf = adj_ref[0].astype(jnp.bfloat16)
        abf_s[pl.ds(row0, _BM), :] = abf
        h = jnp.dot(abf, y1_s[...], preferred_element_type=jnp.float32)
        h1_s[pl.ds(row0, _BM), :] = jnp.maximum(h + b1_ref[m],
                                                0.0).astype(jnp.bfloat16)

    @pl.when(layer == 1)
    def _layer2():
        @pl.when(i == 0)
        def _compute_y2():
            y2_s[...] = jnp.dot(h1_s[...], w2_ref[m],
                                preferred_element_type=jnp.float32
                                ).astype(jnp.bfloat16)

        e = jnp.dot(abf_s[pl.ds(row0, _BM), :], y2_s[...],
                    preferred_element_type=jnp.float32)
        e = jnp.maximum(e + b2_ref[m], 0.0)

        @pl.when(m == 0)
        def _store0():
            e0_s[pl.ds(row0, _BM), :] = e

        @pl.when(m == 1)
        def _store1():
            e1_s[pl.ds(row0, _BM), :] = e

        @pl.when(m == _NMETA - 1)
        def _attend():
            e0 = e0_s[pl.ds(row0, _BM), :]
            e1 = e1_s[pl.ds(row0, _BM), :]
            a_v = a_ref[...]
            s0 = jnp.dot(e0, a_v, preferred_element_type=jnp.float32)
            s1 = jnp.dot(e1, a_v, preferred_element_type=jnp.float32)
            s2 = jnp.dot(e, a_v, preferred_element_type=jnp.float32)
            s0 = jnp.where(s0 > 0, s0, _ALPHA * s0)
            s1 = jnp.where(s1 > 0, s1, _ALPHA * s1)
            s2 = jnp.where(s2 > 0, s2, _ALPHA * s2)
            mx = jnp.maximum(s0, jnp.maximum(s1, s2))
            x0 = jnp.exp(s0 - mx)
            x1 = jnp.exp(s1 - mx)
            x2 = jnp.exp(s2 - mx)
            comb = (x0 * e0 + x1 * e1 + x2 * e) / (x0 + x1 + x2)
            logits = jnp.dot(comb, wlin_ref[...],
                             preferred_element_type=jnp.float32)
            logits = jnp.maximum(logits + blin_ref[...], 0.0)
            zmax = jnp.max(logits, axis=1, keepdims=True)
            z = logits - zmax
            out_ref[...] = z - jnp.log(jnp.sum(jnp.exp(z), axis=1,
                                               keepdims=True))


def kernel(input, adjs, W1, b1, W2, b2, a, Wlin, blin):
    b1r = b1.reshape(_NMETA, 1, _NHID)
    b2r = b2.reshape(_NMETA, 1, _DIM_MP)
    a_r = a.reshape(_DIM_MP, 1)
    blin_r = blin.reshape(1, _NCLASS)
    grid = (_NMETA, 2, _NB)
    return pl.pallas_call(
        _hingcn_body,
        grid=grid,
        in_specs=[
            pl.BlockSpec((_N, _NFEAT), lambda m, l, i: (0, 0)),
            pl.BlockSpec((1, _BM, _N),
                         lambda m, l, i: (m, jnp.where(l == 0, i, _NB - 1),
                                          0)),
            pl.BlockSpec((_NMETA, _NFEAT, _NHID), lambda m, l, i: (0, 0, 0)),
            pl.BlockSpec((_NMETA, 1, _NHID), lambda m, l, i: (0, 0, 0)),
            pl.BlockSpec((_NMETA, _NHID, _DIM_MP), lambda m, l, i: (0, 0, 0)),
            pl.BlockSpec((_NMETA, 1, _DIM_MP), lambda m, l, i: (0, 0, 0)),
            pl.BlockSpec((_DIM_MP, 1), lambda m, l, i: (0, 0)),
            pl.BlockSpec((_DIM_MP, _NCLASS), lambda m, l, i: (0, 0)),
            pl.BlockSpec((1, _NCLASS), lambda m, l, i: (0, 0)),
        ],
        out_specs=pl.BlockSpec((_BM, _NCLASS), lambda m, l, i: (i, 0)),
        out_shape=jax.ShapeDtypeStruct((_N, _NCLASS), jnp.float32),
        compiler_params=pltpu.CompilerParams(
            vmem_limit_bytes=62 * 1024 * 1024,
        ),
        scratch_shapes=[
            pltpu.VMEM((_N, _N), jnp.bfloat16),
            pltpu.VMEM((_N, _NHID), jnp.bfloat16),
            pltpu.VMEM((_N, _NHID), jnp.bfloat16),
            pltpu.VMEM((_N, _DIM_MP), jnp.bfloat16),
            pltpu.VMEM((_N, _DIM_MP), jnp.float32),
            pltpu.VMEM((_N, _DIM_MP), jnp.float32),
        ],
    )(input, adjs, W1, b1r, W2, b2r, a_r, Wlin, blin_r)


# speedup vs baseline: 1.5523x; 1.0788x over previous
import jax
import jax.numpy as jnp
from jax.experimental import pallas as pl
from jax.experimental.pallas import tpu as pltpu

_NFEAT, _NHID, _NMETA, _DIM_MP, _NCLASS = 128, 64, 3, 32, 8
_ALPHA = 0.2
_N = 4096
_BM = 512
_NB = _N // _BM
_BL = 1024
_NL = _N // _BL
_STEPS = _NB + _NL


def _hingcn_body(x_ref, adj_ref, w1_ref, b1_ref, w2_ref, b2_ref, a_ref,
                 wlin_ref, blin_ref, out_ref,
                 abf_s, y1_s, h1_s, y2_s, e0_s, e1_s):
    m = pl.program_id(0)
    j = pl.program_id(1)

    @pl.when(j < _NB)
    def _layer1():
        row0 = j * _BM

        @pl.when(j == 0)
        def _compute_y1():
            y1_s[...] = jnp.dot(x_ref[...], w1_ref[m],
                                preferred_element_type=jnp.float32
                                ).astype(jnp.bfloat16)

        abf = adj_ref[0].astype(jnp.bfloat16)
        abf_s[pl.ds(row0, _BM), :] = abf
        h = jnp.dot(abf, y1_s[...], preferred_element_type=jnp.float32)
        h1_s[pl.ds(row0, _BM), :] = jnp.maximum(h + b1_ref[m],
                                                0.0).astype(jnp.bfloat16)

    @pl.when(j >= _NB)
    def _layer2():
        row0 = (j - _NB) * _BL

        @pl.when(j == _NB)
        def _compute_y2():
            y2_s[...] = jnp.dot(h1_s[...], w2_ref[m],
                                preferred_element_type=jnp.float32
                                ).astype(jnp.bfloat16)

        e = jnp.dot(abf_s[pl.ds(row0, _BL), :], y2_s[...],
                    preferred_element_type=jnp.float32)
        e = jnp.maximum(e + b2_ref[m], 0.0)

        @pl.when(m == 0)
        def _store0():
            e0_s[pl.ds(row0, _BL), :] = e

        @pl.when(m == 1)
        def _store1():
            e1_s[pl.ds(row0, _BL), :] = e

        @pl.when(m == _NMETA - 1)
        def _attend():
            e0 = e0_s[pl.ds(row0, _BL), :]
            e1 = e1_s[pl.ds(row0, _BL), :]
            a_v = a_ref[...]
            s0 = jnp.dot(e0, a_v, preferred_element_type=jnp.float32)
            s1 = jnp.dot(e1, a_v, preferred_element_type=jnp.float32)
            s2 = jnp.dot(e, a_v, preferred_element_type=jnp.float32)
            s0 = jnp.where(s0 > 0, s0, _ALPHA * s0)
            s1 = jnp.where(s1 > 0, s1, _ALPHA * s1)
            s2 = jnp.where(s2 > 0, s2, _ALPHA * s2)
            mx = jnp.maximum(s0, jnp.maximum(s1, s2))
            x0 = jnp.exp(s0 - mx)
            x1 = jnp.exp(s1 - mx)
            x2 = jnp.exp(s2 - mx)
            comb = (x0 * e0 + x1 * e1 + x2 * e) / (x0 + x1 + x2)
            logits = jnp.dot(comb, wlin_ref[...],
                             preferred_element_type=jnp.float32)
            logits = jnp.maximum(logits + blin_ref[...], 0.0)
            zmax = jnp.max(logits, axis=1, keepdims=True)
            z = logits - zmax
            out_ref[...] = z - jnp.log(jnp.sum(jnp.exp(z), axis=1,
                                               keepdims=True))


def kernel(input, adjs, W1, b1, W2, b2, a, Wlin, blin):
    b1r = b1.reshape(_NMETA, 1, _NHID)
    b2r = b2.reshape(_NMETA, 1, _DIM_MP)
    a_r = a.reshape(_DIM_MP, 1)
    blin_r = blin.reshape(1, _NCLASS)
    grid = (_NMETA, _STEPS)
    return pl.pallas_call(
        _hingcn_body,
        grid=grid,
        in_specs=[
            pl.BlockSpec((_N, _NFEAT), lambda m, j: (0, 0)),
            pl.BlockSpec((1, _BM, _N),
                         lambda m, j: (m, jnp.minimum(j, _NB - 1), 0)),
            pl.BlockSpec((_NMETA, _NFEAT, _NHID), lambda m, j: (0, 0, 0)),
            pl.BlockSpec((_NMETA, 1, _NHID), lambda m, j: (0, 0, 0)),
            pl.BlockSpec((_NMETA, _NHID, _DIM_MP), lambda m, j: (0, 0, 0)),
            pl.BlockSpec((_NMETA, 1, _DIM_MP), lambda m, j: (0, 0, 0)),
            pl.BlockSpec((_DIM_MP, 1), lambda m, j: (0, 0)),
            pl.BlockSpec((_DIM_MP, _NCLASS), lambda m, j: (0, 0)),
            pl.BlockSpec((1, _NCLASS), lambda m, j: (0, 0)),
        ],
        out_specs=pl.BlockSpec((_BL, _NCLASS),
                               lambda m, j: (jnp.maximum(j - _NB, 0), 0)),
        out_shape=jax.ShapeDtypeStruct((_N, _NCLASS), jnp.float32),
        compiler_params=pltpu.CompilerParams(
            vmem_limit_bytes=62 * 1024 * 1024,
        ),
        scratch_shapes=[
            pltpu.VMEM((_N, _N), jnp.bfloat16),
            pltpu.VMEM((_N, _NHID), jnp.bfloat16),
            pltpu.VMEM((_N, _NHID), jnp.bfloat16),
            pltpu.VMEM((_N, _DIM_MP), jnp.bfloat16),
            pltpu.VMEM((_N, _DIM_MP), jnp.float32),
            pltpu.VMEM((_N, _DIM_MP), jnp.float32),
        ],
    )(input, adjs, W1, b1r, W2, b2r, a_r, Wlin, blin_r)
